# baseline (speedup 1.0000x reference)
import functools

import jax
import jax.numpy as jnp
from jax import lax
from jax.experimental import pallas as pl
from jax.experimental.pallas import tpu as pltpu
from jax.experimental.pallas import tpu_sc as plsc

NUMEL = 4096 * 4096
K = 524288
NC = 2
NS = 16
NW = NC * NS
C = 32768
NCH = NUMEL // C
CPT = NCH // NW
B = 1024
L = 16
NB = NCH + 8


def _scatter_block(chunk_ref, idx_ref, val_ref, gb, delta):
  for j in range(B // L):
    iv = idx_ref[pl.ds(j * L, L)]
    vv = val_ref[pl.ds(j * L, L)]
    loc = iv - gb
    pos = lax.iota(jnp.int32, L) + (j * L)
    inb = (loc >= 0) & (loc < C) & (pos >= delta)
    lc = jnp.minimum(jnp.maximum(loc, 0), C - 1)
    vz = jnp.where(inb, vv, 0.0)
    plsc.addupdate_scatter(chunk_ref, [lc], vz)


def _body(flat_hbm, idx_hbm, val_hbm, st_hbm, out_hbm,
          st_v, cv0, cv1, ix0, ix1, vl0, vl1,
          isem0, isem1, osem0, osem1, xsem0, xsem1, wsem0, wsem1):
  cid = lax.axis_index("c")
  sid = lax.axis_index("s")
  wid = sid * NC + cid
  cbase = wid * CPT

  bufs = (cv0, cv1)
  ixs = (ix0, ix1)
  vls = (vl0, vl1)
  isems = (isem0, isem1)
  osems = (osem0, osem1)
  xsems = (xsem0, xsem1)
  wsems = (wsem0, wsem1)

  pltpu.sync_copy(st_hbm.at[pl.ds(wid * CPT, 24)], st_v.at[pl.ds(0, 24)])

  def bound_of(c):
    return st_v[pl.ds(c, L)][0]

  def gb_of(c):
    return pl.multiple_of((cbase + c) * C, C)

  def win_of(c):
    s8 = bound_of(c) & -8
    off = jnp.minimum(s8, K - B)
    return pl.multiple_of(off, 8), s8 - off

  def start_in(c, p):
    pltpu.async_copy(flat_hbm.at[pl.ds(gb_of(c), C)], bufs[p], isems[p])
    off, _ = win_of(c)
    pltpu.async_copy(idx_hbm.at[pl.ds(off, B)], ixs[p], xsems[p])
    pltpu.async_copy(val_hbm.at[pl.ds(off, B)], vls[p], wsems[p])

  def wait_in(p):
    pltpu.make_async_copy(flat_hbm.at[pl.ds(0, C)], bufs[p], isems[p]).wait()
    pltpu.make_async_copy(idx_hbm.at[pl.ds(0, B)], ixs[p], xsems[p]).wait()
    pltpu.make_async_copy(val_hbm.at[pl.ds(0, B)], vls[p], wsems[p]).wait()

  def wait_out(p):
    pltpu.make_async_copy(bufs[p], out_hbm.at[pl.ds(0, C)], osems[p]).wait()

  start_in(0, 0)

  def pair_body(g, _):
    for p in (0, 1):
      c = g * 2 + p
      q = 1 - p
      @pl.when(c >= 1)
      def _():
        wait_out(q)

      @pl.when(c + 1 < CPT)
      def _():
        start_in(c + 1, q)

      wait_in(p)

      gb = gb_of(c)
      off0, delta0 = win_of(c)
      end = bound_of(c + 1)
      nb = (end - off0 + (B - 1)) // B

      @pl.when(nb >= 1)
      def _():
        _scatter_block(bufs[p], ixs[p], vls[p], gb, delta0)

      def blk(b, __):
        raw = off0 + b * B
        off = pl.multiple_of(jnp.minimum(raw, K - B), 8)
        pltpu.sync_copy(idx_hbm.at[pl.ds(off, B)], ixs[p])
        pltpu.sync_copy(val_hbm.at[pl.ds(off, B)], vls[p])
        _scatter_block(bufs[p], ixs[p], vls[p], gb, raw - off)
        return 0

      lax.fori_loop(1, nb, blk, 0)
      pltpu.async_copy(bufs[p], out_hbm.at[pl.ds(gb, C)], osems[p])
    return 0

  lax.fori_loop(0, CPT // 2, pair_body, 0)
  wait_out(1)


_sc_call = functools.partial(
    pl.kernel,
    out_type=jax.ShapeDtypeStruct((NUMEL,), jnp.float32),
    mesh=plsc.VectorSubcoreMesh(
        core_axis_name="c", subcore_axis_name="s",
        num_cores=NC, num_subcores=NS),
    compiler_params=pltpu.CompilerParams(needs_layout_passes=False),
    scratch_types=[
        pltpu.VMEM((2 * L,), jnp.int32),
        pltpu.VMEM((C,), jnp.float32),
        pltpu.VMEM((C,), jnp.float32),
        pltpu.VMEM((B,), jnp.int32),
        pltpu.VMEM((B,), jnp.int32),
        pltpu.VMEM((B,), jnp.float32),
        pltpu.VMEM((B,), jnp.float32),
        pltpu.SemaphoreType.DMA,
        pltpu.SemaphoreType.DMA,
        pltpu.SemaphoreType.DMA,
        pltpu.SemaphoreType.DMA,
        pltpu.SemaphoreType.DMA,
        pltpu.SemaphoreType.DMA,
        pltpu.SemaphoreType.DMA,
        pltpu.SemaphoreType.DMA,
    ],
)(_body)


def kernel(tensor, values, indices):
  idx32 = indices.astype(jnp.int32)
  flat = tensor.reshape(-1)
  bounds = jnp.minimum(
      jnp.arange(NB, dtype=jnp.int32) * C, jnp.int32(NUMEL))
  pos = jnp.searchsorted(idx32, bounds, side="left").astype(jnp.int32)
  out = _sc_call(flat, idx32, values, pos)
  return out.reshape(tensor.shape)
